# baseline (speedup 1.0000x reference)
import jax
import jax.numpy as jnp
from jax.experimental import pallas as pl
from jax.experimental.pallas import tpu as pltpu

VOCAB = 32000
EMB = 128
HID = 256
B = 64
T = 32
VC = 3200
NC = VOCAB // VC

_F32 = jnp.float32


def _decoder_body(tok0_ref, emb_ref, wih_ref, whh_ref, bih_ref, bhh_ref,
                  wout_ref, bout_ref, h0_ref,
                  pred_hbm, ptok_ref, dec_ref,
                  h_scr, tok_scr, pred_scr, x_scr, tok_smem, sem, sem_tok):
    t = pl.program_id(0)

    @pl.when(t == 0)
    def _init():
        h_scr[...] = h0_ref[...]
        tok_scr[...] = tok0_ref[...]

    iota = jax.lax.broadcasted_iota(jnp.int32, (B, VC), 1)

    tok_cp = pltpu.make_async_copy(tok_scr, tok_smem, sem_tok)
    tok_cp.start()
    tok_cp.wait()

    for i in range(B):
        ti = tok_smem[i, 0]
        x_scr[i:i + 1, :] = emb_ref[pl.ds(ti, 1), :]
    x = x_scr[...]

    gi = jnp.dot(x, wih_ref[...], preferred_element_type=_F32) + bih_ref[...]
    h = h_scr[...]
    gh = jnp.dot(h, whh_ref[...], preferred_element_type=_F32) + bhh_ref[...]
    r = jax.nn.sigmoid(gi[:, 0:HID] + gh[:, 0:HID])
    z = jax.nn.sigmoid(gi[:, HID:2 * HID] + gh[:, HID:2 * HID])
    n = jnp.tanh(gi[:, 2 * HID:3 * HID] + r * gh[:, 2 * HID:3 * HID])
    h_new = (1.0 - z) * n + z * h
    h_scr[...] = h_new
    dec_ref[0] = h_new

    run_max = jnp.full((B, 1), -jnp.inf, _F32)
    run_idx = jnp.zeros((B, 1), jnp.int32)
    for c in range(NC):
        p = jnp.dot(h_new, wout_ref[:, c * VC:(c + 1) * VC],
                    preferred_element_type=_F32) + bout_ref[:, c * VC:(c + 1) * VC]
        @pl.when(t > 0)
        def _wait_prev(c=c):
            pltpu.make_async_copy(
                pred_scr.at[:, c * VC:(c + 1) * VC],
                pred_hbm.at[t - 1, :, c * VC:(c + 1) * VC],
                sem.at[c],
            ).wait()
        pred_scr[:, c * VC:(c + 1) * VC] = p
        pltpu.make_async_copy(
            pred_scr.at[:, c * VC:(c + 1) * VC],
            pred_hbm.at[t, :, c * VC:(c + 1) * VC],
            sem.at[c],
        ).start()
        cm = jnp.max(p, axis=1, keepdims=True)
        ci = jnp.min(jnp.where(p == cm, iota, VC), axis=1, keepdims=True)
        better = cm > run_max
        run_idx = jnp.where(better, ci + c * VC, run_idx)
        run_max = jnp.where(better, cm, run_max)

    tok_scr[...] = run_idx
    ptok_ref[0] = run_idx

    @pl.when(t == T - 1)
    def _drain():
        for c in range(NC):
            pltpu.make_async_copy(
                pred_scr.at[:, c * VC:(c + 1) * VC],
                pred_hbm.at[t, :, c * VC:(c + 1) * VC],
                sem.at[c],
            ).wait()


@jax.jit
def kernel(outputs, embedding, W_ih, W_hh, b_ih, b_hh, W_out, b_out, h0):
    tok0 = outputs[0].reshape(B, 1).astype(jnp.int32)
    bih2 = b_ih.reshape(1, 3 * HID)
    bhh2 = b_hh.reshape(1, 3 * HID)
    bout2 = b_out.reshape(1, VOCAB)

    const2 = lambda shape: pl.BlockSpec(shape, lambda t: (0, 0))
    preds, ptoks, decs = pl.pallas_call(
        _decoder_body,
        grid=(T,),
        in_specs=[
            const2((B, 1)),
            const2((VOCAB, EMB)),
            const2((EMB, 3 * HID)),
            const2((HID, 3 * HID)),
            const2((1, 3 * HID)),
            const2((1, 3 * HID)),
            const2((HID, VOCAB)),
            const2((1, VOCAB)),
            const2((B, HID)),
        ],
        out_specs=[
            pl.BlockSpec(memory_space=pltpu.MemorySpace.HBM),
            pl.BlockSpec((1, B, 1), lambda t: (t, 0, 0)),
            pl.BlockSpec((1, B, HID), lambda t: (t, 0, 0)),
        ],
        out_shape=[
            jax.ShapeDtypeStruct((T, B, VOCAB), _F32),
            jax.ShapeDtypeStruct((T, B, 1), jnp.int32),
            jax.ShapeDtypeStruct((T, B, HID), _F32),
        ],
        scratch_shapes=[
            pltpu.VMEM((B, HID), _F32),
            pltpu.VMEM((B, 1), jnp.int32),
            pltpu.VMEM((B, VOCAB), _F32),
            pltpu.VMEM((B, EMB), _F32),
            pltpu.SMEM((B, 1), jnp.int32),
            pltpu.SemaphoreType.DMA((NC,)),
            pltpu.SemaphoreType.DMA,
        ],
        compiler_params=pltpu.CompilerParams(
            dimension_semantics=("arbitrary",),
            vmem_limit_bytes=63 * 1024 * 1024,
        ),
    )(tok0, embedding, W_ih, W_hh, bih2, bhh2, W_out, bout2, h0)

    return preds, ptoks.reshape(T, B), decs


# speedup vs baseline: 1.3306x; 1.3306x over previous
import jax
import jax.numpy as jnp
from jax.experimental import pallas as pl
from jax.experimental.pallas import tpu as pltpu

VOCAB = 32000
EMB = 128
HID = 256
B = 64
T = 32
VC = 3200
NC = VOCAB // VC

_F32 = jnp.float32


def _decoder_body(tok0_ref, emb_ref, wih_ref, whh_ref, bih_ref, bhh_ref,
                  wout_ref, bout_ref, h0_ref,
                  pred_hbm, ptok_ref, dec_ref,
                  h_scr, tok_scr, pred_scr, x_scr, tok_smem, sem, sem_tok):
    t = pl.program_id(0)

    @pl.when(t == 0)
    def _init():
        h_scr[...] = h0_ref[...]
        tok_scr[...] = tok0_ref[...]
        pltpu.make_async_copy(tok_scr, tok_smem, sem_tok).start()

    iota = jax.lax.broadcasted_iota(jnp.int32, (B, VC), 1)

    pltpu.make_async_copy(tok_scr, tok_smem, sem_tok).wait()

    @pl.when(t > 0)
    def _wait_prev():
        for c in range(NC):
            pltpu.make_async_copy(
                pred_scr.at[:, c * VC:(c + 1) * VC],
                pred_hbm.at[t - 1, :, c * VC:(c + 1) * VC],
                sem.at[c],
            ).wait()

    for i in range(B):
        ti = tok_smem[i, 0]
        x_scr[i:i + 1, :] = emb_ref[pl.ds(ti, 1), :]
    x = x_scr[...]

    gi = jnp.dot(x, wih_ref[...], preferred_element_type=_F32) + bih_ref[...]
    h = h_scr[...]
    gh = jnp.dot(h, whh_ref[...], preferred_element_type=_F32) + bhh_ref[...]
    r = jax.nn.sigmoid(gi[:, 0:HID] + gh[:, 0:HID])
    z = jax.nn.sigmoid(gi[:, HID:2 * HID] + gh[:, HID:2 * HID])
    n = jnp.tanh(gi[:, 2 * HID:3 * HID] + r * gh[:, 2 * HID:3 * HID])
    h_new = (1.0 - z) * n + z * h
    h_scr[...] = h_new
    dec_ref[0] = h_new

    run_max = jnp.full((B, 1), -jnp.inf, _F32)
    run_idx = jnp.zeros((B, 1), jnp.int32)
    for c in range(NC):
        p = jnp.dot(h_new, wout_ref[:, c * VC:(c + 1) * VC],
                    preferred_element_type=_F32) + bout_ref[:, c * VC:(c + 1) * VC]
        pred_scr[:, c * VC:(c + 1) * VC] = p
        pltpu.make_async_copy(
            pred_scr.at[:, c * VC:(c + 1) * VC],
            pred_hbm.at[t, :, c * VC:(c + 1) * VC],
            sem.at[c],
        ).start()
        cm = jnp.max(p, axis=1, keepdims=True)
        ci = jnp.min(jnp.where(p == cm, iota, VC), axis=1, keepdims=True)
        better = cm > run_max
        run_idx = jnp.where(better, ci + c * VC, run_idx)
        run_max = jnp.where(better, cm, run_max)

    tok_scr[...] = run_idx
    ptok_ref[0] = run_idx

    @pl.when(t < T - 1)
    def _tok_ship():
        pltpu.make_async_copy(tok_scr, tok_smem, sem_tok).start()

    @pl.when(t == T - 1)
    def _drain():
        for c in range(NC):
            pltpu.make_async_copy(
                pred_scr.at[:, c * VC:(c + 1) * VC],
                pred_hbm.at[t, :, c * VC:(c + 1) * VC],
                sem.at[c],
            ).wait()


@jax.jit
def kernel(outputs, embedding, W_ih, W_hh, b_ih, b_hh, W_out, b_out, h0):
    tok0 = outputs[0].reshape(B, 1).astype(jnp.int32)
    bih2 = b_ih.reshape(1, 3 * HID)
    bhh2 = b_hh.reshape(1, 3 * HID)
    bout2 = b_out.reshape(1, VOCAB)

    const2 = lambda shape: pl.BlockSpec(shape, lambda t: (0, 0))
    preds, ptoks, decs = pl.pallas_call(
        _decoder_body,
        grid=(T,),
        in_specs=[
            const2((B, 1)),
            const2((VOCAB, EMB)),
            const2((EMB, 3 * HID)),
            const2((HID, 3 * HID)),
            const2((1, 3 * HID)),
            const2((1, 3 * HID)),
            const2((HID, VOCAB)),
            const2((1, VOCAB)),
            const2((B, HID)),
        ],
        out_specs=[
            pl.BlockSpec(memory_space=pltpu.MemorySpace.HBM),
            pl.BlockSpec((1, B, 1), lambda t: (t, 0, 0)),
            pl.BlockSpec((1, B, HID), lambda t: (t, 0, 0)),
        ],
        out_shape=[
            jax.ShapeDtypeStruct((T, B, VOCAB), _F32),
            jax.ShapeDtypeStruct((T, B, 1), jnp.int32),
            jax.ShapeDtypeStruct((T, B, HID), _F32),
        ],
        scratch_shapes=[
            pltpu.VMEM((B, HID), _F32),
            pltpu.VMEM((B, 1), jnp.int32),
            pltpu.VMEM((B, VOCAB), _F32),
            pltpu.VMEM((B, EMB), _F32),
            pltpu.SMEM((B, 1), jnp.int32),
            pltpu.SemaphoreType.DMA((NC,)),
            pltpu.SemaphoreType.DMA,
        ],
        compiler_params=pltpu.CompilerParams(
            dimension_semantics=("arbitrary",),
            vmem_limit_bytes=63 * 1024 * 1024,
        ),
    )(tok0, embedding, W_ih, W_hh, bih2, bhh2, W_out, bout2, h0)

    return preds, ptoks.reshape(T, B), decs


# speedup vs baseline: 1.3425x; 1.0089x over previous
import jax
import jax.numpy as jnp
from jax.experimental import pallas as pl
from jax.experimental.pallas import tpu as pltpu

VOCAB = 32000
EMB = 128
HID = 256
B = 64
T = 32
VC = 3200
NC = VOCAB // VC

_F32 = jnp.float32


def _decoder_body(tok0_ref, emb_ref, wih_ref, whh_ref, bih_ref, bhh_ref,
                  wout_ref, bout_ref, h0_ref,
                  pred_hbm, ptok_hbm, dec_hbm,
                  h_scr, tok_scr, pred_scr, x_scr, tok_smem,
                  sem, sem_tok, sem_dec, sem_ptok):
    t = pl.program_id(0)

    @pl.when(t == 0)
    def _init():
        h_scr[...] = h0_ref[...]
        tok_scr[...] = tok0_ref[...]
        pltpu.make_async_copy(tok_scr, tok_smem, sem_tok).start()

    iota = jax.lax.broadcasted_iota(jnp.int32, (B, VC), 1)

    pltpu.make_async_copy(tok_scr, tok_smem, sem_tok).wait()

    @pl.when(t > 0)
    def _wait_prev():
        for c in range(NC):
            pltpu.make_async_copy(
                pred_scr.at[:, c * VC:(c + 1) * VC],
                pred_hbm.at[t - 1, :, c * VC:(c + 1) * VC],
                sem.at[c],
            ).wait()

    for i in range(B):
        ti = tok_smem[i, 0]
        x_scr[i:i + 1, :] = emb_ref[pl.ds(ti, 1), :]
    x = x_scr[...]

    gi = jnp.dot(x, wih_ref[...], preferred_element_type=_F32) + bih_ref[...]
    h = h_scr[...]
    gh = jnp.dot(h, whh_ref[...], preferred_element_type=_F32) + bhh_ref[...]
    r = jax.nn.sigmoid(gi[:, 0:HID] + gh[:, 0:HID])
    z = jax.nn.sigmoid(gi[:, HID:2 * HID] + gh[:, HID:2 * HID])
    n = jnp.tanh(gi[:, 2 * HID:3 * HID] + r * gh[:, 2 * HID:3 * HID])
    h_new = (1.0 - z) * n + z * h

    @pl.when(t > 0)
    def _wait_dec():
        pltpu.make_async_copy(h_scr, dec_hbm.at[t - 1], sem_dec).wait()
    h_scr[...] = h_new
    pltpu.make_async_copy(h_scr, dec_hbm.at[t], sem_dec).start()

    run_max = jnp.full((B, 1), -jnp.inf, _F32)
    run_idx = jnp.zeros((B, 1), jnp.int32)
    for c in range(NC):
        p = jnp.dot(h_new, wout_ref[:, c * VC:(c + 1) * VC],
                    preferred_element_type=_F32) + bout_ref[:, c * VC:(c + 1) * VC]
        pred_scr[:, c * VC:(c + 1) * VC] = p
        pltpu.make_async_copy(
            pred_scr.at[:, c * VC:(c + 1) * VC],
            pred_hbm.at[t, :, c * VC:(c + 1) * VC],
            sem.at[c],
        ).start()
        cm = jnp.max(p, axis=1, keepdims=True)
        ci = jnp.min(jnp.where(p == cm, iota, VC), axis=1, keepdims=True)
        better = cm > run_max
        run_idx = jnp.where(better, ci + c * VC, run_idx)
        run_max = jnp.where(better, cm, run_max)

    @pl.when(t > 0)
    def _wait_ptok():
        pltpu.make_async_copy(tok_scr, ptok_hbm.at[t - 1], sem_ptok).wait()
    tok_scr[...] = run_idx
    pltpu.make_async_copy(tok_scr, ptok_hbm.at[t], sem_ptok).start()

    @pl.when(t < T - 1)
    def _tok_ship():
        pltpu.make_async_copy(tok_scr, tok_smem, sem_tok).start()

    @pl.when(t == T - 1)
    def _drain():
        for c in range(NC):
            pltpu.make_async_copy(
                pred_scr.at[:, c * VC:(c + 1) * VC],
                pred_hbm.at[t, :, c * VC:(c + 1) * VC],
                sem.at[c],
            ).wait()
        pltpu.make_async_copy(h_scr, dec_hbm.at[t], sem_dec).wait()
        pltpu.make_async_copy(tok_scr, ptok_hbm.at[t], sem_ptok).wait()


@jax.jit
def kernel(outputs, embedding, W_ih, W_hh, b_ih, b_hh, W_out, b_out, h0):
    tok0 = outputs[0].reshape(B, 1).astype(jnp.int32)
    bih2 = b_ih.reshape(1, 3 * HID)
    bhh2 = b_hh.reshape(1, 3 * HID)
    bout2 = b_out.reshape(1, VOCAB)

    const2 = lambda shape: pl.BlockSpec(shape, lambda t: (0, 0))
    preds, ptoks, decs = pl.pallas_call(
        _decoder_body,
        grid=(T,),
        in_specs=[
            const2((B, 1)),
            const2((VOCAB, EMB)),
            const2((EMB, 3 * HID)),
            const2((HID, 3 * HID)),
            const2((1, 3 * HID)),
            const2((1, 3 * HID)),
            const2((HID, VOCAB)),
            const2((1, VOCAB)),
            const2((B, HID)),
        ],
        out_specs=[
            pl.BlockSpec(memory_space=pltpu.MemorySpace.HBM),
            pl.BlockSpec(memory_space=pltpu.MemorySpace.HBM),
            pl.BlockSpec(memory_space=pltpu.MemorySpace.HBM),
        ],
        out_shape=[
            jax.ShapeDtypeStruct((T, B, VOCAB), _F32),
            jax.ShapeDtypeStruct((T, B, 1), jnp.int32),
            jax.ShapeDtypeStruct((T, B, HID), _F32),
        ],
        scratch_shapes=[
            pltpu.VMEM((B, HID), _F32),
            pltpu.VMEM((B, 1), jnp.int32),
            pltpu.VMEM((B, VOCAB), _F32),
            pltpu.VMEM((B, EMB), _F32),
            pltpu.SMEM((B, 1), jnp.int32),
            pltpu.SemaphoreType.DMA((NC,)),
            pltpu.SemaphoreType.DMA,
            pltpu.SemaphoreType.DMA,
            pltpu.SemaphoreType.DMA,
        ],
        compiler_params=pltpu.CompilerParams(
            dimension_semantics=("arbitrary",),
            vmem_limit_bytes=63 * 1024 * 1024,
        ),
    )(tok0, embedding, W_ih, W_hh, bih2, bhh2, W_out, bout2, h0)

    return preds, ptoks.reshape(T, B), decs
